# baseline (speedup 1.0000x reference)
import functools

import jax
import jax.numpy as jnp
from jax import lax
from jax.experimental import pallas as pl
from jax.experimental.pallas import tpu as pltpu
from jax.experimental.pallas import tpu_sc as plsc

CAP = 4096
DIM = 128
BATCH = 1024
MOM = 0.9
LR = 0.1

_NC = 2
_NS = 16
_NW = _NC * _NS
_L = 16
_UPD_ROWS = BATCH // _NW
_CPY_ROWS = (CAP - BATCH) // _NW


def _sc_body(x_hbm, mem_hbm, mom_hbm, out_hbm, x_v, mem_v, mom_v, cpy_v,
             sem_in, sem_cp, sem_out):
    wid = lax.axis_index("s") * _NC + lax.axis_index("c")
    ubase = wid * _UPD_ROWS
    cbase = BATCH + wid * _CPY_ROWS

    cpy_in = pltpu.async_copy(mem_hbm.at[pl.ds(cbase, _CPY_ROWS)], cpy_v, sem_cp)
    ld_x = pltpu.async_copy(x_hbm.at[pl.ds(ubase, _UPD_ROWS)], x_v, sem_in)
    ld_me = pltpu.async_copy(mem_hbm.at[pl.ds(ubase, _UPD_ROWS)], mem_v, sem_in)
    ld_mo = pltpu.async_copy(mom_hbm.at[pl.ds(ubase, _UPD_ROWS)], mom_v, sem_in)

    cpy_in.wait()
    cpy_out = pltpu.async_copy(cpy_v, out_hbm.at[pl.ds(cbase, _CPY_ROWS)], sem_out)

    ld_x.wait()
    ld_me.wait()
    ld_mo.wait()

    @plsc.parallel_loop(0, _UPD_ROWS, 1, unroll=2)
    def row(i):
        for j in range(DIM // _L):
            sl = pl.ds(j * _L, _L)
            m = mem_v[i, sl]
            new_mom = MOM * mom_v[i, sl] + (1.0 - MOM) * (x_v[i, sl] - m)
            mem_v[i, sl] = m + LR * new_mom

    upd_out = pltpu.async_copy(mem_v, out_hbm.at[pl.ds(ubase, _UPD_ROWS)], sem_out)
    cpy_out.wait()
    upd_out.wait()


@functools.cache
def _titans_sc():
    return pl.kernel(
        _sc_body,
        out_type=jax.ShapeDtypeStruct((CAP, DIM), jnp.float32),
        mesh=plsc.VectorSubcoreMesh(core_axis_name="c", subcore_axis_name="s"),
        scratch_types=[
            pltpu.VMEM((_UPD_ROWS, DIM), jnp.float32),
            pltpu.VMEM((_UPD_ROWS, DIM), jnp.float32),
            pltpu.VMEM((_UPD_ROWS, DIM), jnp.float32),
            pltpu.VMEM((_CPY_ROWS, DIM), jnp.float32),
            pltpu.SemaphoreType.DMA,
            pltpu.SemaphoreType.DMA,
            pltpu.SemaphoreType.DMA,
        ],
    )


def kernel(x, memory, surprise_scores, momentum_buffer):
    del surprise_scores
    return _titans_sc()(x, memory, momentum_buffer)


# speedup vs baseline: 1.0341x; 1.0341x over previous
import functools

import jax
import jax.numpy as jnp
from jax import lax
from jax.experimental import pallas as pl
from jax.experimental.pallas import tpu as pltpu
from jax.experimental.pallas import tpu_sc as plsc

CAP = 4096
DIM = 128
BATCH = 1024
MOM = 0.9
LR = 0.1

_NC = 2
_NS = 16
_NW = _NC * _NS
_L = 16
_UPD_ROWS = BATCH // _NW
_CPY_ROWS = (CAP - BATCH) // _NW


def _sc_body(x_hbm, mem_hbm, mom_hbm, out_hbm, x_v, mem_v, mom_v, cpy_v,
             sem_in, sem_cp, sem_out):
    wid = lax.axis_index("s") * _NC + lax.axis_index("c")
    ubase = wid * _UPD_ROWS
    cbase = BATCH + wid * _CPY_ROWS

    cpy_in = pltpu.async_copy(mem_hbm.at[pl.ds(cbase, _CPY_ROWS)], cpy_v, sem_cp)
    ld_x = pltpu.async_copy(x_hbm.at[pl.ds(ubase, _UPD_ROWS)], x_v, sem_in)
    ld_me = pltpu.async_copy(mem_hbm.at[pl.ds(ubase, _UPD_ROWS)], mem_v, sem_in)
    ld_mo = pltpu.async_copy(mom_hbm.at[pl.ds(ubase, _UPD_ROWS)], mom_v, sem_in)

    cpy_in.wait()
    cpy_out = pltpu.async_copy(cpy_v, out_hbm.at[pl.ds(cbase, _CPY_ROWS)], sem_out)

    ld_x.wait()
    ld_me.wait()
    ld_mo.wait()

    def row(i, carry):
        for j in range(DIM // _L):
            sl = pl.ds(j * _L, _L)
            m = mem_v[i, sl]
            new_mom = MOM * mom_v[i, sl] + (1.0 - MOM) * (x_v[i, sl] - m)
            mem_v[i, sl] = m + LR * new_mom
        return carry

    lax.fori_loop(0, _UPD_ROWS, row, 0)

    upd_out = pltpu.async_copy(mem_v, out_hbm.at[pl.ds(ubase, _UPD_ROWS)], sem_out)
    cpy_out.wait()
    upd_out.wait()


@functools.cache
def _titans_sc():
    return pl.kernel(
        _sc_body,
        out_type=jax.ShapeDtypeStruct((CAP, DIM), jnp.float32),
        mesh=plsc.VectorSubcoreMesh(core_axis_name="c", subcore_axis_name="s"),
        scratch_types=[
            pltpu.VMEM((_UPD_ROWS, DIM), jnp.float32),
            pltpu.VMEM((_UPD_ROWS, DIM), jnp.float32),
            pltpu.VMEM((_UPD_ROWS, DIM), jnp.float32),
            pltpu.VMEM((_CPY_ROWS, DIM), jnp.float32),
            pltpu.SemaphoreType.DMA,
            pltpu.SemaphoreType.DMA,
            pltpu.SemaphoreType.DMA,
        ],
    )


def kernel(x, memory, surprise_scores, momentum_buffer):
    del surprise_scores
    return _titans_sc()(x, memory, momentum_buffer)
